# baseline (speedup 1.0000x reference)
import functools

import jax
import jax.numpy as jnp
from jax import lax
from jax.experimental import pallas as pl
from jax.experimental.pallas import tpu as pltpu
from jax.experimental.pallas import tpu_sc as plsc

NUM_CORES = 2
NUM_SUBCORES = 16
NUM_WORKERS = NUM_CORES * NUM_SUBCORES
LANES = 16

CHUNK = 64
IBLK = 8
NBUF = 2


def _encoder_call(ex_ids2, ex_par, cat_ids3, ex_table2, cat_table2, pos_pk):
  n_blocks, iblk, chunk = ex_ids2.shape
  dim = 64
  n_chunks = n_blocks * iblk
  seq_len = 2 * pos_pk.shape[0]
  assert chunk == CHUNK and iblk == IBLK and seq_len == 200
  assert n_chunks % NUM_WORKERS == 0
  chunks_per_w = n_chunks // NUM_WORKERS
  blocks_per_w = chunks_per_w // IBLK
  n_rows = n_chunks * chunk
  n_cat_pairs = cat_table2.shape[0]

  mesh = plsc.VectorSubcoreMesh(
      core_axis_name="c", subcore_axis_name="s",
      num_cores=NUM_CORES, num_subcores=NUM_SUBCORES)

  @functools.partial(
      pl.kernel,
      out_type=jax.ShapeDtypeStruct((n_rows // 2, 2 * dim), jnp.float32),
      mesh=mesh,
      compiler_params=pltpu.CompilerParams(use_tc_tiling_on_sc=True),
      scratch_types=[
          [pltpu.VMEM((CHUNK, 128), jnp.float32)] * NBUF,
          [pltpu.VMEM((CHUNK // 2, 128), jnp.float32)] * NBUF,
          [pltpu.VMEM((IBLK, CHUNK), jnp.int32)] * 2,
          [pltpu.VMEM((IBLK, CHUNK), jnp.int32)] * 2,
          [pltpu.VMEM((IBLK, CHUNK), jnp.int32)] * 2,
          pltpu.VMEM((n_cat_pairs, 128), jnp.float32),
          pltpu.VMEM((seq_len // 2, 128), jnp.float32),
          [pltpu.SemaphoreType.DMA] * NBUF,
          [pltpu.SemaphoreType.DMA] * NBUF,
          pltpu.SemaphoreType.DMA,
      ],
  )
  def enc(ex_ids_hbm, par_hbm, cat_ids_hbm, ex_table_hbm, cat_table_hbm,
          pos_hbm, out_hbm, ex_bufs, out_bufs, exi, pari, cati, cat_v, pos_v,
          gsems, ssems, isem):
    wid = lax.axis_index("s") * NUM_CORES + lax.axis_index("c")
    blk0 = wid * blocks_per_w
    g0 = wid * chunks_per_w

    pltpu.sync_copy(pos_hbm, pos_v)
    pltpu.sync_copy(cat_table_hbm, cat_v)
    pltpu.sync_copy(ex_ids_hbm.at[blk0], exi[0])
    pltpu.sync_copy(par_hbm.at[blk0], pari[0])
    pltpu.sync_copy(cat_ids_hbm.at[blk0], cati[0])

    def issue_gather(exi_ref, row, slot):
      pltpu.async_copy(ex_table_hbm.at[exi_ref.at[row]], ex_bufs[slot],
                       gsems[slot])

    for slot in range(NBUF):
      issue_gather(exi[0], slot, slot)

    def compute(u, jrow, eb, ob, par_ref, cat_ref):
      r0 = lax.rem(u * CHUNK, seq_len)
      c0 = r0 // 2
      w = jnp.minimum(seq_len - r0, CHUNK)
      wh = w // 2

      def group(g, carry):
        p0 = LANES * g
        par16 = par_ref[jrow, pl.ds(p0, LANES)]
        cat16 = cat_ref[jrow, pl.ds(p0, LANES)]
        for hh in range(2):
          h = 2 * g + hh
          rb = jnp.where(8 * h < w, c0 + 4 * h, 4 * h - wh)
          for j in range(8):
            k = 8 * hh + j
            p = p0 + k
            oe = par16[k]
            c = cat16[k]
            cr = c // 2
            cc = (c % 2) * dim
            prow = rb + j // 2
            ph = (j % 2) * dim
            orow = p0 // 2 + k // 2
            oc = (k % 2) * dim
            for q in range(dim // LANES):
              ob[orow, pl.ds(oc + q * LANES, LANES)] = (
                  eb[p, pl.ds(oe + q * LANES, LANES)]
                  + cat_v[cr, pl.ds(cc + q * LANES, LANES)]
                  + pos_v[prow, pl.ds(ph + q * LANES, LANES)])
        return carry
      lax.fori_loop(0, CHUNK // LANES, group, 0)

    def do_block(blk, cur, nxt):
      exi_cur, pari_cur, cati_cur = cur
      exi_nxt, pari_nxt, cati_nxt = nxt

      @pl.when(blk + 1 < blocks_per_w)
      def _():
        pltpu.async_copy(ex_ids_hbm.at[blk0 + blk + 1], exi_nxt, isem)
        pltpu.async_copy(par_hbm.at[blk0 + blk + 1], pari_nxt, isem)
        pltpu.async_copy(cat_ids_hbm.at[blk0 + blk + 1], cati_nxt, isem)

      def inner(jj, carry):
        for s in range(NBUF):
          jrow = 2 * jj + s
          u = blk * IBLK + jrow
          pltpu.make_async_copy(
              ex_table_hbm.at[pl.ds(0, CHUNK)], ex_bufs[s], gsems[s]).wait()

          @pl.when(u >= NBUF)
          def _():
            pltpu.make_async_copy(
                out_bufs[s], out_hbm.at[pl.ds(0, CHUNK // 2)], ssems[s]).wait()

          compute(u, jrow, ex_bufs[s], out_bufs[s], pari_cur, cati_cur)

          @pl.when(jj < IBLK // 2 - 1)
          def _():
            issue_gather(exi_cur, jrow + 2, s)

          @pl.when((jj == IBLK // 2 - 1) & (blk + 1 < blocks_per_w))
          def _():
            if s == 0:
              for ref in (exi_nxt, pari_nxt, cati_nxt):
                pltpu.make_async_copy(
                    ex_ids_hbm.at[blk0], ref, isem).wait()
            issue_gather(exi_nxt, s, s)

          pltpu.async_copy(
              out_bufs[s],
              out_hbm.at[pl.ds((g0 + u) * (CHUNK // 2), CHUNK // 2)],
              ssems[s])
        return carry
      lax.fori_loop(0, IBLK // NBUF, inner, 0)

    def outer(blk, carry):
      @pl.when(lax.rem(blk, 2) == 0)
      def _():
        do_block(blk, (exi[0], pari[0], cati[0]), (exi[1], pari[1], cati[1]))

      @pl.when(lax.rem(blk, 2) == 1)
      def _():
        do_block(blk, (exi[1], pari[1], cati[1]), (exi[0], pari[0], cati[0]))
      return carry
    lax.fori_loop(0, blocks_per_w, outer, 0)

    for slot in range(NBUF):
      pltpu.make_async_copy(
          out_bufs[slot], out_hbm.at[pl.ds(0, CHUNK // 2)], ssems[slot]).wait()

  return enc(ex_ids2, ex_par, cat_ids3, ex_table2, cat_table2, pos_pk)


def kernel(exercise_ids, category_ids, exercise_table, category_table,
           positional_table):
  batch, seq_len = exercise_ids.shape
  dim = exercise_table.shape[1]
  ex_ids = exercise_ids.astype(jnp.int32)
  ex_ids2 = (ex_ids // 2).reshape(-1, IBLK, CHUNK)
  ex_par = ((ex_ids % 2) * dim).reshape(-1, IBLK, CHUNK)
  cat_ids3 = category_ids.astype(jnp.int32).reshape(-1, IBLK, CHUNK)
  ex_table2 = exercise_table.astype(jnp.float32).reshape(-1, 2 * dim)
  cat_table2 = category_table.astype(jnp.float32).reshape(-1, 2 * dim)
  pos_pk = positional_table.astype(jnp.float32).reshape(seq_len // 2, 2 * dim)
  out = _encoder_call(ex_ids2, ex_par, cat_ids3, ex_table2, cat_table2,
                      pos_pk)
  return out.reshape(batch, seq_len, dim)


# speedup vs baseline: 1.2057x; 1.2057x over previous
import functools

import jax
import jax.numpy as jnp
from jax import lax
from jax.experimental import pallas as pl
from jax.experimental.pallas import tpu as pltpu
from jax.experimental.pallas import tpu_sc as plsc

NUM_CORES = 2
NUM_SUBCORES = 16
NUM_WORKERS = NUM_CORES * NUM_SUBCORES
LANES = 16

CHUNK = 64
IBLK = 8
NBUF = 2


def _encoder_call(ex_ids2, ex_par, cat_ids3, ex_table2, cat_table2, pos_pk):
  n_blocks, iblk, chunk = ex_ids2.shape
  dim = 64
  n_chunks = n_blocks * iblk
  seq_len = 2 * pos_pk.shape[0]
  assert chunk == CHUNK and iblk == IBLK and seq_len == 200
  assert n_chunks % NUM_WORKERS == 0
  chunks_per_w = n_chunks // NUM_WORKERS
  blocks_per_w = chunks_per_w // IBLK
  n_rows = n_chunks * chunk
  n_cat_pairs = cat_table2.shape[0]

  mesh = plsc.VectorSubcoreMesh(
      core_axis_name="c", subcore_axis_name="s",
      num_cores=NUM_CORES, num_subcores=NUM_SUBCORES)

  @functools.partial(
      pl.kernel,
      out_type=jax.ShapeDtypeStruct((n_rows, dim), jnp.float32),
      mesh=mesh,
      compiler_params=pltpu.CompilerParams(use_tc_tiling_on_sc=True),
      scratch_types=[
          [pltpu.VMEM((CHUNK, 128), jnp.float32)] * NBUF,
          [pltpu.VMEM((CHUNK, dim), jnp.float32)] * NBUF,
          [pltpu.VMEM((IBLK, CHUNK), jnp.int32)] * 2,
          [pltpu.VMEM((IBLK, CHUNK), jnp.int32)] * 2,
          [pltpu.VMEM((IBLK, CHUNK), jnp.int32)] * 2,
          pltpu.VMEM((n_cat_pairs, 128), jnp.float32),
          pltpu.VMEM((seq_len // 2, 128), jnp.float32),
          [pltpu.SemaphoreType.DMA] * NBUF,
          [pltpu.SemaphoreType.DMA] * NBUF,
          pltpu.SemaphoreType.DMA,
      ],
  )
  def enc(ex_ids_hbm, par_hbm, cat_ids_hbm, ex_table_hbm, cat_table_hbm,
          pos_hbm, out_hbm, ex_bufs, out_bufs, exi, pari, cati, cat_v, pos_v,
          gsems, ssems, isem):
    wid = lax.axis_index("s") * NUM_CORES + lax.axis_index("c")
    blk0 = wid * blocks_per_w
    g0 = wid * chunks_per_w

    pltpu.sync_copy(pos_hbm, pos_v)
    pltpu.sync_copy(cat_table_hbm, cat_v)
    pltpu.sync_copy(ex_ids_hbm.at[blk0], exi[0])
    pltpu.sync_copy(par_hbm.at[blk0], pari[0])
    pltpu.sync_copy(cat_ids_hbm.at[blk0], cati[0])

    def issue_gather(exi_ref, row, slot):
      pltpu.async_copy(ex_table_hbm.at[exi_ref.at[row]], ex_bufs[slot],
                       gsems[slot])

    for slot in range(NBUF):
      issue_gather(exi[0], slot, slot)

    def compute(u, jrow, eb, ob, par_ref, cat_ref):
      r0 = lax.rem(u * CHUNK, seq_len)
      c0 = r0 // 2
      w = jnp.minimum(seq_len - r0, CHUNK)
      wh = w // 2

      def group(g, carry):
        p0 = LANES * g
        par16 = par_ref[jrow, pl.ds(p0, LANES)]
        cat16 = cat_ref[jrow, pl.ds(p0, LANES)]
        for hh in range(2):
          h = 2 * g + hh
          rb = jnp.where(8 * h < w, c0 + 4 * h, 4 * h - wh)
          for j in range(8):
            k = 8 * hh + j
            p = p0 + k
            oe = par16[k]
            c = cat16[k]
            cr = c // 2
            cc = (c % 2) * dim
            prow = rb + j // 2
            ph = (j % 2) * dim
            for q in range(dim // LANES):
              ob[p, pl.ds(q * LANES, LANES)] = (
                  eb[p, pl.ds(oe + q * LANES, LANES)]
                  + cat_v[cr, pl.ds(cc + q * LANES, LANES)]
                  + pos_v[prow, pl.ds(ph + q * LANES, LANES)])
        return carry
      lax.fori_loop(0, CHUNK // LANES, group, 0)

    def do_block(blk, cur, nxt):
      exi_cur, pari_cur, cati_cur = cur
      exi_nxt, pari_nxt, cati_nxt = nxt

      @pl.when(blk + 1 < blocks_per_w)
      def _():
        pltpu.async_copy(ex_ids_hbm.at[blk0 + blk + 1], exi_nxt, isem)
        pltpu.async_copy(par_hbm.at[blk0 + blk + 1], pari_nxt, isem)
        pltpu.async_copy(cat_ids_hbm.at[blk0 + blk + 1], cati_nxt, isem)

      def inner(jj, carry):
        for s in range(NBUF):
          jrow = 2 * jj + s
          u = blk * IBLK + jrow
          pltpu.make_async_copy(
              ex_table_hbm.at[pl.ds(0, CHUNK)], ex_bufs[s], gsems[s]).wait()

          @pl.when(u >= NBUF)
          def _():
            pltpu.make_async_copy(
                out_bufs[s], out_hbm.at[pl.ds(0, CHUNK)], ssems[s]).wait()

          compute(u, jrow, ex_bufs[s], out_bufs[s], pari_cur, cati_cur)

          @pl.when(jj < IBLK // 2 - 1)
          def _():
            issue_gather(exi_cur, jrow + 2, s)

          @pl.when((jj == IBLK // 2 - 1) & (blk + 1 < blocks_per_w))
          def _():
            if s == 0:
              for ref in (exi_nxt, pari_nxt, cati_nxt):
                pltpu.make_async_copy(
                    ex_ids_hbm.at[blk0], ref, isem).wait()
            issue_gather(exi_nxt, s, s)

          pltpu.async_copy(
              out_bufs[s],
              out_hbm.at[pl.ds((g0 + u) * CHUNK, CHUNK)],
              ssems[s])
        return carry
      lax.fori_loop(0, IBLK // NBUF, inner, 0)

    def outer(blk, carry):
      @pl.when(lax.rem(blk, 2) == 0)
      def _():
        do_block(blk, (exi[0], pari[0], cati[0]), (exi[1], pari[1], cati[1]))

      @pl.when(lax.rem(blk, 2) == 1)
      def _():
        do_block(blk, (exi[1], pari[1], cati[1]), (exi[0], pari[0], cati[0]))
      return carry
    lax.fori_loop(0, blocks_per_w, outer, 0)

    for slot in range(NBUF):
      pltpu.make_async_copy(
          out_bufs[slot], out_hbm.at[pl.ds(0, CHUNK)], ssems[slot]).wait()

  return enc(ex_ids2, ex_par, cat_ids3, ex_table2, cat_table2, pos_pk)


def kernel(exercise_ids, category_ids, exercise_table, category_table,
           positional_table):
  batch, seq_len = exercise_ids.shape
  dim = exercise_table.shape[1]
  ex_ids = exercise_ids.astype(jnp.int32)
  ex_ids2 = (ex_ids // 2).reshape(-1, IBLK, CHUNK)
  ex_par = ((ex_ids % 2) * dim).reshape(-1, IBLK, CHUNK)
  cat_ids3 = category_ids.astype(jnp.int32).reshape(-1, IBLK, CHUNK)
  ex_table2 = exercise_table.astype(jnp.float32).reshape(-1, 2 * dim)
  cat_table2 = category_table.astype(jnp.float32).reshape(-1, 2 * dim)
  pos_pk = positional_table.astype(jnp.float32).reshape(seq_len // 2, 2 * dim)
  out = _encoder_call(ex_ids2, ex_par, cat_ids3, ex_table2, cat_table2,
                      pos_pk)
  return out.reshape(batch, seq_len, dim)
